# Initial kernel scaffold; baseline (speedup 1.0000x reference)
import jax
import jax.numpy as jnp
from jax.experimental import pallas as pl


def kernel(preds, feat1, feat2, feat3, feat4, Wa, ba, Wb, bb):
    raise NotImplementedError("write your pallas kernel here")


# speedup vs baseline: 2.3004x; 2.3004x over previous
import jax
import jax.numpy as jnp
from jax.experimental import pallas as pl
from jax.experimental.pallas import tpu as pltpu

_CONF = 0.596
_IOU = 0.45
_NDET = 300
_NPAD = 304
_NCAND = 20000


def _score_kernel(x_ref, meta_ref):
    x = x_ref[...]
    obj = x[:, 4:5]
    cls_s = x[:, 5:85] * obj
    conf = jnp.max(cls_s, axis=1, keepdims=True)
    li = jax.lax.broadcasted_iota(jnp.int32, cls_s.shape, 1)
    cls = jnp.min(jnp.where(cls_s == conf, li, 80), axis=1, keepdims=True)
    valid = (obj > _CONF) & (conf > _CONF)
    masked = jnp.where(valid, conf, -1.0)
    cxcy = x[:, 0:2]
    wh = x[:, 2:4]
    meta_ref[...] = jnp.concatenate(
        [cxcy - wh * 0.5, cxcy + wh * 0.5, cls.astype(jnp.float32), masked],
        axis=1,
    )


def _topk_kernel(s_in_ref, meta_ref, sel_ref, s_ref):
    s_ref[...] = s_in_ref[...]
    ri = jax.lax.broadcasted_iota(jnp.int32, (8, 2500), 0)
    ci = jax.lax.broadcasted_iota(jnp.int32, (8, 2500), 1)
    flat = ri * 2500 + ci
    rsel = jax.lax.broadcasted_iota(jnp.int32, (_NPAD, 6), 0)
    sel_ref[...] = jnp.zeros((_NPAD, 6), jnp.float32)

    def body(i, carry):
        s = s_ref[...]
        m = jnp.max(s)
        idx = jnp.min(jnp.where(s == m, flat, _NCAND))
        s_ref[...] = jnp.where(flat == idx, -2.0, s)
        row = meta_ref[pl.ds(idx, 1), :]
        sel_ref[...] = jnp.where(rsel == i,
                                 jnp.broadcast_to(row, (_NPAD, 6)),
                                 sel_ref[...])
        return carry

    jax.lax.fori_loop(0, _NDET, body, 0)


def _nms_kernel(selT_ref, keep_ref):
    off = selT_ref[4:5, :] * 4096.0
    x1 = selT_ref[0:1, :] + off
    y1 = selT_ref[1:2, :] + off
    x2 = selT_ref[2:3, :] + off
    y2 = selT_ref[3:4, :] + off
    sc = selT_ref[5:6, :]
    area = (x2 - x1) * (y2 - y1)
    ji = jax.lax.broadcasted_iota(jnp.int32, (1, _NPAD), 1)
    keep_ref[...] = (sc > 0.0).astype(jnp.float32)

    def body(i, carry):
        keep = keep_ref[...]
        m = (ji == i).astype(jnp.float32)
        bx1 = jnp.sum(x1 * m)
        by1 = jnp.sum(y1 * m)
        bx2 = jnp.sum(x2 * m)
        by2 = jnp.sum(y2 * m)
        barea = jnp.sum(area * m)
        ki = jnp.sum(keep * m)
        iw = jnp.clip(jnp.minimum(bx2, x2) - jnp.maximum(bx1, x1), 0.0, None)
        ih = jnp.clip(jnp.minimum(by2, y2) - jnp.maximum(by1, y1), 0.0, None)
        inter = iw * ih
        iou = inter / (barea + area - inter + 1e-9)
        sup = (iou > _IOU) & (ji > i) & (ki > 0.5)
        keep_ref[...] = jnp.where(sup, 0.0, keep)
        return carry

    jax.lax.fori_loop(0, _NDET, body, 0)


def _roi_kernel(sel_ref, f1_ref, f2_ref, f3_ref, f4_ref, out_ref):
    levels = (
        (f1_ref, 1.0 / 8, 96, 96, 128, 0),
        (f2_ref, 1.0 / 16, 48, 48, 256, 128),
        (f3_ref, 1.0 / 32, 24, 24, 512, 384),
        (f4_ref, 1.0 / 64, 12, 12, 1024, 896),
    )
    for j in range(8):
        bx1 = jnp.sum(sel_ref[j:j + 1, 0:1])
        by1 = jnp.sum(sel_ref[j:j + 1, 1:2])
        bx2 = jnp.sum(sel_ref[j:j + 1, 2:3])
        by2 = jnp.sum(sel_ref[j:j + 1, 3:4])
        for fref, s, H, W, C, off in levels:
            x1 = bx1 * s
            y1 = by1 * s
            x2 = bx2 * s
            y2 = by2 * s
            xA = x1 + 0.25 * (x2 - x1) - 0.5
            xB = x1 + 0.75 * (x2 - x1) - 0.5
            yA = y1 + 0.25 * (y2 - y1) - 0.5
            yB = y1 + 0.75 * (y2 - y1) - 0.5
            acc = jnp.zeros((1, C), jnp.float32)
            for yy, xx in ((yA, xA), (yA, xB), (yB, xA), (yB, xB)):
                y = jnp.clip(yy, 0.0, H - 1.0)
                x = jnp.clip(xx, 0.0, W - 1.0)
                y0f = jnp.floor(y)
                x0f = jnp.floor(x)
                y0 = y0f.astype(jnp.int32)
                x0 = x0f.astype(jnp.int32)
                y1i = jnp.minimum(y0 + 1, H - 1)
                x1i = jnp.minimum(x0 + 1, W - 1)
                ly = y - y0f
                lx = x - x0f
                r00 = fref[pl.ds(y0 * W + x0, 1), :]
                r01 = fref[pl.ds(y0 * W + x1i, 1), :]
                r10 = fref[pl.ds(y1i * W + x0, 1), :]
                r11 = fref[pl.ds(y1i * W + x1i, 1), :]
                acc = acc + (r00 * ((1.0 - ly) * (1.0 - lx))
                             + r01 * ((1.0 - ly) * lx)
                             + r10 * (ly * (1.0 - lx))
                             + r11 * (ly * lx))
            out_ref[j:j + 1, off:off + C] = acc * 0.25


def _mlp_kernel(f_ref, sel_ref, keep_ref, wa_ref, ba_ref, wb_ref, bb_ref,
                out_ref):
    f = f_ref[...]
    h = jnp.dot(f, wa_ref[...], preferred_element_type=jnp.float32) + ba_ref[...]
    h = jnp.where(h >= 0, h, 0.01 * h)
    h = jnp.dot(h, wb_ref[...], preferred_element_type=jnp.float32) + bb_ref[...]
    h = jnp.where(h >= 0, h, 0.01 * h)
    keep = keep_ref[...]
    nb = sel_ref[:, 0:4] * (1.0 / 96.0)
    out_ref[...] = jnp.concatenate([nb, h], axis=1) * keep


def kernel(preds, feat1, feat2, feat3, feat4, Wa, ba, Wb, bb):
    x = preds[0]
    meta = pl.pallas_call(
        _score_kernel,
        grid=(10,),
        in_specs=[pl.BlockSpec((2000, 85), lambda i: (i, 0))],
        out_specs=pl.BlockSpec((2000, 6), lambda i: (i, 0)),
        out_shape=jax.ShapeDtypeStruct((_NCAND, 6), jnp.float32),
    )(x)
    s2d = meta[:, 5].reshape(8, 2500)
    sel = pl.pallas_call(
        _topk_kernel,
        out_shape=jax.ShapeDtypeStruct((_NPAD, 6), jnp.float32),
        scratch_shapes=[pltpu.VMEM((8, 2500), jnp.float32)],
    )(s2d, meta)
    keep = pl.pallas_call(
        _nms_kernel,
        out_shape=jax.ShapeDtypeStruct((1, _NPAD), jnp.float32),
    )(sel.T)
    f1t = feat1[0].reshape(128, 96 * 96).T
    f2t = feat2[0].reshape(256, 48 * 48).T
    f3t = feat3[0].reshape(512, 24 * 24).T
    f4t = feat4[0].reshape(1024, 12 * 12).T
    froi = pl.pallas_call(
        _roi_kernel,
        grid=(_NPAD // 8,),
        in_specs=[
            pl.BlockSpec((8, 6), lambda k: (k, 0)),
            pl.BlockSpec((96 * 96, 128), lambda k: (0, 0)),
            pl.BlockSpec((48 * 48, 256), lambda k: (0, 0)),
            pl.BlockSpec((24 * 24, 512), lambda k: (0, 0)),
            pl.BlockSpec((12 * 12, 1024), lambda k: (0, 0)),
        ],
        out_specs=pl.BlockSpec((8, 1920), lambda k: (k, 0)),
        out_shape=jax.ShapeDtypeStruct((_NPAD, 1920), jnp.float32),
    )(sel, f1t, f2t, f3t, f4t)
    out = pl.pallas_call(
        _mlp_kernel,
        out_shape=jax.ShapeDtypeStruct((_NPAD, 68), jnp.float32),
    )(froi, sel, keep.T, Wa, ba.reshape(1, 64), Wb, bb.reshape(1, 64))
    return out[:_NDET]
